# baseline (speedup 1.0000x reference)
import functools

import jax
import jax.numpy as jnp
import numpy as np
from jax.experimental import pallas as pl
from jax.experimental.pallas import tpu as pltpu

SEQ = 2048
D = 1024
NH = 16
DH = 64
DFF = 4096
IH = 4
ID = 64
TOPK = 512
EPS = 1e-6

BQ = 128
NBQ = SEQ // BQ
NCOLS = 3 * D + IH * ID + ID + IH
NPAD = 3456

_MININT = np.int32(-2147483648)


def _rms(x, g):
    n = jnp.sqrt(jnp.sum(x * x, axis=-1, keepdims=True)) * (x.shape[-1] ** -0.5)
    return g * (x / (n + EPS))


def _proj_kernel(x_ref, g_ref, w_ref, p_ref):
    h = _rms(x_ref[...], g_ref[...])
    p_ref[...] = jnp.dot(h, w_ref[...], preferred_element_type=jnp.float32)


_NT = (((1,), (1,)), ((), ()))


def _attn_body(strip_ref, k_ref, v_ref, ki_ref, o_ref, qb, width):
    ki = ki_ref[:width, :ID]
    acc = jnp.zeros((BQ, width), jnp.float32)
    for h in range(IH):
        qih = strip_ref[:, 3 * D + h * ID:3 * D + (h + 1) * ID]
        d = jax.lax.dot_general(qih, ki, _NT,
                                preferred_element_type=jnp.float32)
        wc = 3 * D + IH * ID + ID + h
        wcol = strip_ref[:, wc:wc + 1]
        acc = acc + wcol * jnp.maximum(d, 0.0)
    col = jax.lax.broadcasted_iota(jnp.int32, (BQ, width), 1)
    row = qb * BQ + jax.lax.broadcasted_iota(jnp.int32, (BQ, width), 0)
    causal = col <= row
    b = jax.lax.bitcast_convert_type(acc, jnp.int32)
    keys = b ^ ((b >> 31) & np.int32(0x7FFFFFFF))
    keys = jnp.where(causal, keys, _MININT)
    g = jnp.zeros((BQ, 1), jnp.int32)
    for i in range(31, -1, -1):
        bit = np.uint32(1 << i).astype(np.int32)
        trial = g | bit
        cnt = jnp.sum((keys >= (trial ^ _MININT)).astype(jnp.int32),
                      axis=1, keepdims=True)
        g = jnp.where(cnt >= TOPK, trial, g)
    tau = g ^ _MININT
    cnt_gt = jnp.sum((keys > tau).astype(jnp.int32), axis=1, keepdims=True)
    needed = TOPK - cnt_gt
    tie = keys == tau
    cp = jnp.zeros((BQ, 1), jnp.int32)
    nbits = max(1, (width - 1).bit_length())
    for i in range(nbits, -1, -1):
        trial = cp + np.int32(1 << i)
        cnt = jnp.sum((tie & (col < trial)).astype(jnp.int32),
                      axis=1, keepdims=True)
        cp = jnp.where(cnt <= needed, trial, cp)
    sel = (keys > tau) | (tie & (col < cp))
    bias = jnp.where(sel & causal, 0.0, -jnp.inf)
    scale = DH ** -0.5
    for h in range(NH):
        qh = strip_ref[:, h * DH:(h + 1) * DH] * scale
        logits = jax.lax.dot_general(
            qh, k_ref[:width, h * DH:(h + 1) * DH], _NT,
            preferred_element_type=jnp.float32)
        logits = logits + bias
        m = jnp.max(logits, axis=1, keepdims=True)
        p = jnp.exp(logits - m)
        s = jnp.sum(p, axis=1, keepdims=True)
        o_ref[:, h * DH:(h + 1) * DH] = jnp.dot(
            p, v_ref[:width, h * DH:(h + 1) * DH],
            preferred_element_type=jnp.float32) / s


def _attn_kernel(strip_ref, k_ref, v_ref, ki_ref, o_ref, *, qb0, width):
    _attn_body(strip_ref, k_ref, v_ref, ki_ref, o_ref,
               qb0 + pl.program_id(0), width)


def _attn_call(P, qb0, nq, width):
    return pl.pallas_call(
        functools.partial(_attn_kernel, qb0=qb0, width=width),
        grid=(nq,),
        in_specs=[
            pl.BlockSpec((BQ, NPAD), lambda i: (qb0 + i, 0)),
            pl.BlockSpec((width, D), lambda i: (0, 1)),
            pl.BlockSpec((width, D), lambda i: (0, 2)),
            pl.BlockSpec((width, 128), lambda i: (0, (3 * D + IH * ID) // 128)),
        ],
        out_specs=pl.BlockSpec((BQ, D), lambda i: (i, 0)),
        out_shape=jax.ShapeDtypeStruct((nq * BQ, D), jnp.float32),
        compiler_params=pltpu.CompilerParams(
            dimension_semantics=("arbitrary",),
            vmem_limit_bytes=60 * 1024 * 1024,
        ),
    )(P, P, P, P)


def _ffn_kernel(ctx_ref, x_ref, g_ref, wo_ref, w1_ref, w2_ref, o_ref):
    bf = jnp.bfloat16
    y = x_ref[...] + jnp.dot(ctx_ref[...].astype(bf), wo_ref[...],
                             preferred_element_type=jnp.float32)
    h2 = _rms(y, g_ref[...])
    f = jnp.dot(h2.astype(bf), w1_ref[...],
                preferred_element_type=jnp.float32)
    f = f * 0.5 * (1.0 + jax.lax.erf(f * np.float32(2.0 ** -0.5)))
    o_ref[...] = y + jnp.dot(f.astype(bf), w2_ref[...],
                             preferred_element_type=jnp.float32)


@jax.jit
def kernel(x, g1, g2, Wq, Wk, Wv, Wo, Wqi, Wki, Wwi, W1, W2):
    x2 = x.reshape(SEQ, D)
    wcat = jnp.concatenate([Wq, Wk, Wv, Wqi, Wki, Wwi], axis=1)
    wcat = jnp.pad(wcat, ((0, 0), (0, NPAD - NCOLS)))
    P = pl.pallas_call(
        _proj_kernel,
        grid=(NBQ,),
        in_specs=[
            pl.BlockSpec((BQ, D), lambda i: (i, 0)),
            pl.BlockSpec((1, D), lambda i: (0, 0)),
            pl.BlockSpec((D, NPAD), lambda i: (0, 0)),
        ],
        out_specs=pl.BlockSpec((BQ, NPAD), lambda i: (i, 0)),
        out_shape=jax.ShapeDtypeStruct((SEQ, NPAD), jnp.float32),
        compiler_params=pltpu.CompilerParams(
            dimension_semantics=("arbitrary",),
            vmem_limit_bytes=60 * 1024 * 1024,
        ),
    )(x2, g1.reshape(1, D), wcat)

    ctx = jnp.concatenate(
        [_attn_call(P, 4 * w, 4, 512 * (w + 1)) for w in range(4)], axis=0)

    out = pl.pallas_call(
        _ffn_kernel,
        grid=(NBQ,),
        in_specs=[
            pl.BlockSpec((BQ, D), lambda i: (i, 0)),
            pl.BlockSpec((BQ, D), lambda i: (i, 0)),
            pl.BlockSpec((1, D), lambda i: (0, 0)),
            pl.BlockSpec((D, D), lambda i: (0, 0)),
            pl.BlockSpec((D, DFF), lambda i: (0, 0)),
            pl.BlockSpec((DFF, D), lambda i: (0, 0)),
        ],
        out_specs=pl.BlockSpec((BQ, D), lambda i: (i, 0)),
        out_shape=jax.ShapeDtypeStruct((SEQ, D), jnp.float32),
        compiler_params=pltpu.CompilerParams(
            dimension_semantics=("arbitrary",),
            vmem_limit_bytes=60 * 1024 * 1024,
        ),
    )(ctx, x2, g2.reshape(1, D), Wo.astype(jnp.bfloat16),
      W1.astype(jnp.bfloat16), W2.astype(jnp.bfloat16))

    return out.reshape(1, SEQ, D)


# speedup vs baseline: 1.0382x; 1.0382x over previous
import functools

import jax
import jax.numpy as jnp
import numpy as np
from jax.experimental import pallas as pl
from jax.experimental.pallas import tpu as pltpu

SEQ = 2048
D = 1024
NH = 16
DH = 64
DFF = 4096
IH = 4
ID = 64
TOPK = 512
EPS = 1e-6

BQ = 128
NBQ = SEQ // BQ
NCOLS = 3 * D + IH * ID + ID + IH
NPAD = 3456

_MININT = np.int32(-2147483648)


def _rms(x, g):
    n = jnp.sqrt(jnp.sum(x * x, axis=-1, keepdims=True)) * (x.shape[-1] ** -0.5)
    return g * (x / (n + EPS))


def _proj_kernel(x_ref, g_ref, w_ref, p_ref):
    h = _rms(x_ref[...], g_ref[...])
    p_ref[...] = jnp.dot(h, w_ref[...], preferred_element_type=jnp.float32)


_NT = (((1,), (1,)), ((), ()))


def _attn_body(strip_ref, k_ref, v_ref, ki_ref, o_ref, qb, width):
    ki = ki_ref[:width, :ID]
    acc = jnp.zeros((BQ, width), jnp.float32)
    for h in range(IH):
        qih = strip_ref[:, 3 * D + h * ID:3 * D + (h + 1) * ID]
        d = jax.lax.dot_general(qih, ki, _NT,
                                preferred_element_type=jnp.float32)
        wc = 3 * D + IH * ID + ID + h
        wcol = strip_ref[:, wc:wc + 1]
        acc = acc + wcol * jnp.maximum(d, 0.0)
    col = jax.lax.broadcasted_iota(jnp.int32, (BQ, width), 1)
    row = qb * BQ + jax.lax.broadcasted_iota(jnp.int32, (BQ, width), 0)
    causal = col <= row
    b = jax.lax.bitcast_convert_type(acc, jnp.int32)
    keys = b ^ ((b >> 31) & np.int32(0x7FFFFFFF))
    keys = jnp.where(causal, keys, _MININT)
    g = jnp.zeros((BQ, 1), jnp.int32)
    for i in range(31, -1, -1):
        bit = np.uint32(1 << i).astype(np.int32)
        trial = g | bit
        cnt = jnp.sum((keys >= (trial ^ _MININT)).astype(jnp.int32),
                      axis=1, keepdims=True)
        g = jnp.where(cnt >= TOPK, trial, g)
    tau = g ^ _MININT
    cnt_gt = jnp.sum((keys > tau).astype(jnp.int32), axis=1, keepdims=True)
    needed = TOPK - cnt_gt
    tie = keys == tau
    cp = jnp.zeros((BQ, 1), jnp.int32)
    nbits = max(1, (width - 1).bit_length())
    for i in range(nbits, -1, -1):
        trial = cp + np.int32(1 << i)
        cnt = jnp.sum((tie & (col < trial)).astype(jnp.int32),
                      axis=1, keepdims=True)
        cp = jnp.where(cnt <= needed, trial, cp)
    sel = (keys > tau) | (tie & (col < cp))
    bias = jnp.where(sel & causal, 0.0, -jnp.inf)
    scale = DH ** -0.5
    for h in range(NH):
        qh = strip_ref[:, h * DH:(h + 1) * DH] * scale
        logits = jax.lax.dot_general(
            qh, k_ref[:width, h * DH:(h + 1) * DH], _NT,
            preferred_element_type=jnp.float32)
        logits = logits + bias
        m = jnp.max(logits, axis=1, keepdims=True)
        p = jnp.exp(logits - m)
        s = jnp.sum(p, axis=1, keepdims=True)
        o_ref[:, h * DH:(h + 1) * DH] = jnp.dot(
            p, v_ref[:width, h * DH:(h + 1) * DH],
            preferred_element_type=jnp.float32) / s


def _attn_kernel(strip_ref, k_ref, v_ref, ki_ref, o_ref, *, qb0, width):
    _attn_body(strip_ref, k_ref, v_ref, ki_ref, o_ref,
               qb0 + pl.program_id(0), width)


def _attn_call(P, qb0, nq, width):
    return pl.pallas_call(
        functools.partial(_attn_kernel, qb0=qb0, width=width),
        grid=(nq,),
        in_specs=[
            pl.BlockSpec((BQ, NPAD), lambda i: (qb0 + i, 0)),
            pl.BlockSpec((width, D), lambda i: (0, 1)),
            pl.BlockSpec((width, D), lambda i: (0, 2)),
            pl.BlockSpec((width, 128), lambda i: (0, (3 * D + IH * ID) // 128)),
        ],
        out_specs=pl.BlockSpec((BQ, D), lambda i: (i, 0)),
        out_shape=jax.ShapeDtypeStruct((nq * BQ, D), jnp.float32),
        compiler_params=pltpu.CompilerParams(
            dimension_semantics=("parallel",),
            vmem_limit_bytes=60 * 1024 * 1024,
        ),
    )(P, P, P, P)


def _ffn_kernel(ctx_ref, x_ref, g_ref, wo_ref, w1_ref, w2_ref, o_ref):
    y = x_ref[...] + jnp.dot(ctx_ref[...], wo_ref[...],
                             preferred_element_type=jnp.float32)
    h2 = _rms(y, g_ref[...])
    f = jnp.dot(h2, w1_ref[...], preferred_element_type=jnp.float32)
    f = f * 0.5 * (1.0 + jax.lax.erf(f * np.float32(2.0 ** -0.5)))
    o_ref[...] = y + jnp.dot(f, w2_ref[...],
                             preferred_element_type=jnp.float32)


@jax.jit
def kernel(x, g1, g2, Wq, Wk, Wv, Wo, Wqi, Wki, Wwi, W1, W2):
    x2 = x.reshape(SEQ, D)
    wcat = jnp.concatenate([Wq, Wk, Wv, Wqi, Wki, Wwi], axis=1)
    wcat = jnp.pad(wcat, ((0, 0), (0, NPAD - NCOLS)))
    P = pl.pallas_call(
        _proj_kernel,
        grid=(NBQ,),
        in_specs=[
            pl.BlockSpec((BQ, D), lambda i: (i, 0)),
            pl.BlockSpec((1, D), lambda i: (0, 0)),
            pl.BlockSpec((D, NPAD), lambda i: (0, 0)),
        ],
        out_specs=pl.BlockSpec((BQ, NPAD), lambda i: (i, 0)),
        out_shape=jax.ShapeDtypeStruct((SEQ, NPAD), jnp.float32),
        compiler_params=pltpu.CompilerParams(
            dimension_semantics=("parallel",),
            vmem_limit_bytes=60 * 1024 * 1024,
        ),
    )(x2, g1.reshape(1, D), wcat)

    ctx = jnp.concatenate(
        [_attn_call(P, 4 * w, 4, 512 * (w + 1)) for w in range(4)], axis=0)

    out = pl.pallas_call(
        _ffn_kernel,
        grid=(NBQ,),
        in_specs=[
            pl.BlockSpec((BQ, D), lambda i: (i, 0)),
            pl.BlockSpec((BQ, D), lambda i: (i, 0)),
            pl.BlockSpec((1, D), lambda i: (0, 0)),
            pl.BlockSpec((D, D), lambda i: (0, 0)),
            pl.BlockSpec((D, DFF), lambda i: (0, 0)),
            pl.BlockSpec((DFF, D), lambda i: (0, 0)),
        ],
        out_specs=pl.BlockSpec((BQ, D), lambda i: (i, 0)),
        out_shape=jax.ShapeDtypeStruct((SEQ, D), jnp.float32),
        compiler_params=pltpu.CompilerParams(
            dimension_semantics=("parallel",),
            vmem_limit_bytes=60 * 1024 * 1024,
        ),
    )(ctx, x2, g2.reshape(1, D), Wo, W1, W2)

    return out.reshape(1, SEQ, D)


# speedup vs baseline: 1.1602x; 1.1175x over previous
import functools

import jax
import jax.numpy as jnp
import numpy as np
from jax.experimental import pallas as pl
from jax.experimental.pallas import tpu as pltpu

SEQ = 2048
D = 1024
NH = 16
DH = 64
DFF = 4096
IH = 4
ID = 64
TOPK = 512
EPS = 1e-6

BQ = 256
NBQ = SEQ // BQ
NCOLS = 3 * D + IH * ID + ID + IH
NPAD = 3456

_MININT = np.int32(-2147483648)


def _rms(x, g):
    n = jnp.sqrt(jnp.sum(x * x, axis=-1, keepdims=True)) * (x.shape[-1] ** -0.5)
    return g * (x / (n + EPS))


def _proj_kernel(x_ref, g_ref, w_ref, p_ref):
    h = _rms(x_ref[...], g_ref[...])
    p_ref[...] = jnp.dot(h, w_ref[...], preferred_element_type=jnp.float32)


_NT = (((1,), (1,)), ((), ()))


def _attn_body(strip_ref, k_ref, v_ref, ki_ref, o_ref, qb, width):
    ki = ki_ref[:width, :ID]
    acc = jnp.zeros((BQ, width), jnp.float32)
    for h in range(IH):
        qih = strip_ref[:, 3 * D + h * ID:3 * D + (h + 1) * ID]
        d = jax.lax.dot_general(qih, ki, _NT,
                                preferred_element_type=jnp.float32)
        wc = 3 * D + IH * ID + ID + h
        wcol = strip_ref[:, wc:wc + 1]
        acc = acc + wcol * jnp.maximum(d, 0.0)
    col = jax.lax.broadcasted_iota(jnp.int32, (BQ, width), 1)
    row = qb * BQ + jax.lax.broadcasted_iota(jnp.int32, (BQ, width), 0)
    causal = col <= row
    b = jax.lax.bitcast_convert_type(acc, jnp.int32)
    keys = b ^ ((b >> 31) & np.int32(0x7FFFFFFF))
    keys = jnp.where(causal, keys, _MININT)
    g = jnp.zeros((BQ, 1), jnp.int32)
    for i in range(31, -1, -1):
        bit = np.uint32(1 << i).astype(np.int32)
        trial = g | bit
        cnt = jnp.sum((keys >= (trial ^ _MININT)).astype(jnp.int32),
                      axis=1, keepdims=True)
        g = jnp.where(cnt >= TOPK, trial, g)
    tau = g ^ _MININT
    cnt_gt = jnp.sum((keys > tau).astype(jnp.int32), axis=1, keepdims=True)
    needed = TOPK - cnt_gt
    tie = keys == tau
    cp = jnp.zeros((BQ, 1), jnp.int32)
    nbits = max(1, (width - 1).bit_length())
    for i in range(nbits, -1, -1):
        trial = cp + np.int32(1 << i)
        cnt = jnp.sum((tie & (col < trial)).astype(jnp.int32),
                      axis=1, keepdims=True)
        cp = jnp.where(cnt <= needed, trial, cp)
    sel = (keys > tau) | (tie & (col < cp))
    bias = jnp.where(sel & causal, 0.0, -jnp.inf)
    scale = DH ** -0.5
    for h in range(NH):
        qh = strip_ref[:, h * DH:(h + 1) * DH] * scale
        logits = jax.lax.dot_general(
            qh, k_ref[:width, h * DH:(h + 1) * DH], _NT,
            preferred_element_type=jnp.float32)
        logits = logits + bias
        m = jnp.max(logits, axis=1, keepdims=True)
        p = jnp.exp(logits - m)
        s = jnp.sum(p, axis=1, keepdims=True)
        o_ref[:, h * DH:(h + 1) * DH] = jnp.dot(
            p, v_ref[:width, h * DH:(h + 1) * DH],
            preferred_element_type=jnp.float32) / s


def _attn_kernel(strip_ref, k_ref, v_ref, ki_ref, o_ref, *, qb0, width):
    _attn_body(strip_ref, k_ref, v_ref, ki_ref, o_ref,
               qb0 + pl.program_id(0), width)


def _attn_call(P, qb0, nq, width):
    return pl.pallas_call(
        functools.partial(_attn_kernel, qb0=qb0, width=width),
        grid=(nq,),
        in_specs=[
            pl.BlockSpec((BQ, NPAD), lambda i: (qb0 + i, 0)),
            pl.BlockSpec((width, D), lambda i: (0, 1)),
            pl.BlockSpec((width, D), lambda i: (0, 2)),
            pl.BlockSpec((width, 128), lambda i: (0, (3 * D + IH * ID) // 128)),
        ],
        out_specs=pl.BlockSpec((BQ, D), lambda i: (i, 0)),
        out_shape=jax.ShapeDtypeStruct((nq * BQ, D), jnp.float32),
        compiler_params=pltpu.CompilerParams(
            dimension_semantics=("parallel",),
            vmem_limit_bytes=60 * 1024 * 1024,
        ),
    )(P, P, P, P)


def _ffn_kernel(ctx_ref, x_ref, g_ref, wo_ref, w1_ref, w2_ref, o_ref):
    y = x_ref[...] + jnp.dot(ctx_ref[...], wo_ref[...],
                             preferred_element_type=jnp.float32)
    h2 = _rms(y, g_ref[...])
    f = jnp.dot(h2, w1_ref[...], preferred_element_type=jnp.float32)
    f = f * 0.5 * (1.0 + jax.lax.erf(f * np.float32(2.0 ** -0.5)))
    o_ref[...] = y + jnp.dot(f, w2_ref[...],
                             preferred_element_type=jnp.float32)


@jax.jit
def kernel(x, g1, g2, Wq, Wk, Wv, Wo, Wqi, Wki, Wwi, W1, W2):
    x2 = x.reshape(SEQ, D)
    wcat = jnp.concatenate([Wq, Wk, Wv, Wqi, Wki, Wwi], axis=1)
    wcat = jnp.pad(wcat, ((0, 0), (0, NPAD - NCOLS)))
    P = pl.pallas_call(
        _proj_kernel,
        grid=(NBQ,),
        in_specs=[
            pl.BlockSpec((BQ, D), lambda i: (i, 0)),
            pl.BlockSpec((1, D), lambda i: (0, 0)),
            pl.BlockSpec((D, NPAD), lambda i: (0, 0)),
        ],
        out_specs=pl.BlockSpec((BQ, NPAD), lambda i: (i, 0)),
        out_shape=jax.ShapeDtypeStruct((SEQ, NPAD), jnp.float32),
        compiler_params=pltpu.CompilerParams(
            dimension_semantics=("parallel",),
            vmem_limit_bytes=60 * 1024 * 1024,
        ),
    )(x2, g1.reshape(1, D), wcat)

    npw = 512 // BQ
    ctx = jnp.concatenate(
        [_attn_call(P, npw * w, npw, 512 * (w + 1)) for w in range(4)], axis=0)

    out = pl.pallas_call(
        _ffn_kernel,
        grid=(NBQ,),
        in_specs=[
            pl.BlockSpec((BQ, D), lambda i: (i, 0)),
            pl.BlockSpec((BQ, D), lambda i: (i, 0)),
            pl.BlockSpec((1, D), lambda i: (0, 0)),
            pl.BlockSpec((D, D), lambda i: (0, 0)),
            pl.BlockSpec((D, DFF), lambda i: (0, 0)),
            pl.BlockSpec((DFF, D), lambda i: (0, 0)),
        ],
        out_specs=pl.BlockSpec((BQ, D), lambda i: (i, 0)),
        out_shape=jax.ShapeDtypeStruct((SEQ, D), jnp.float32),
        compiler_params=pltpu.CompilerParams(
            dimension_semantics=("parallel",),
            vmem_limit_bytes=60 * 1024 * 1024,
        ),
    )(ctx, x2, g2.reshape(1, D), Wo, W1, W2)

    return out.reshape(1, SEQ, D)
